# baseline (speedup 1.0000x reference)
import jax
import jax.numpy as jnp
from jax import lax
from jax.experimental import pallas as pl
from jax.experimental.pallas import tpu as pltpu
from jax.experimental.pallas import tpu_sc as plsc

_VOCAB = 1000000
_DIM = 64
_B = 1024
_L = 200

_HALF = 100
_ROWS = _B * _L // _HALF
_NW = 32
_CPW = _ROWS // _NW
_LANES = 16


def _pe_table():
    pos = jnp.arange(_L, dtype=jnp.float32)[:, None]
    i = jnp.arange(0, _DIM, 2, dtype=jnp.float32)
    div = jnp.exp(-jnp.log(10000.0) * i / _DIM)
    angles = pos * div[None, :]
    pe = jnp.zeros((_L, _DIM), dtype=jnp.float32)
    pe = pe.at[:, 0::2].set(jnp.sin(angles))
    pe = pe.at[:, 1::2].set(jnp.cos(angles))
    return pe


def _sc_body(inp_hbm, pe_hbm, table_hbm, out_hbm, idx_v, pe_v, rows_v, gsem):
    wid = lax.axis_index("s") * 2 + lax.axis_index("c")
    base = wid * _CPW
    pltpu.sync_copy(inp_hbm.at[pl.ds(base, _CPW)], idx_v)
    pltpu.sync_copy(pe_hbm, pe_v)

    def chunk_group(g, carry):
        for p in range(2):
            j = g * 2 + p
            pltpu.async_copy(table_hbm.at[idx_v.at[j]], rows_v, gsem).wait()

            def add_body(r, c):
                for d in range(_DIM // _LANES):
                    sl = pl.ds(d * _LANES, _LANES)
                    rows_v[r, sl] = rows_v[r, sl] + pe_v[p * _HALF + r, sl]
                return c

            lax.fori_loop(0, _HALF, add_body, 0, unroll=4)
            pltpu.sync_copy(rows_v, out_hbm.at[base + j])
        return carry

    lax.fori_loop(0, _CPW // 2, chunk_group, 0)


def kernel(inp, table):
    pe = _pe_table()
    inp2 = inp.reshape(_ROWS, _HALF).astype(jnp.int32)
    mesh = plsc.VectorSubcoreMesh(core_axis_name="c", subcore_axis_name="s")
    run = pl.kernel(
        _sc_body,
        out_type=jax.ShapeDtypeStruct((_ROWS, _HALF, _DIM), jnp.float32),
        mesh=mesh,
        scratch_types=[
            pltpu.VMEM((_CPW, _HALF), jnp.int32),
            pltpu.VMEM((_L, _DIM), jnp.float32),
            pltpu.VMEM((_HALF, _DIM), jnp.float32),
            pltpu.SemaphoreType.DMA,
        ],
        compiler_params=pltpu.CompilerParams(use_tc_tiling_on_sc=False),
    )
    out = run(inp2, pe, table)
    return out.reshape(_B, _L, _DIM)


# speedup vs baseline: 1.1022x; 1.1022x over previous
import jax
import jax.numpy as jnp
from jax import lax
from jax.experimental import pallas as pl
from jax.experimental.pallas import tpu as pltpu
from jax.experimental.pallas import tpu_sc as plsc

_VOCAB = 1000000
_DIM = 64
_B = 1024
_L = 200

_HALF = 100
_ROWS = _B * _L // _HALF
_NW = 32
_CPW = _ROWS // _NW
_LANES = 16


def _pe_table():
    pos = jnp.arange(_L, dtype=jnp.float32)[:, None]
    i = jnp.arange(0, _DIM, 2, dtype=jnp.float32)
    div = jnp.exp(-jnp.log(10000.0) * i / _DIM)
    angles = pos * div[None, :]
    pe = jnp.zeros((_L, _DIM), dtype=jnp.float32)
    pe = pe.at[:, 0::2].set(jnp.sin(angles))
    pe = pe.at[:, 1::2].set(jnp.cos(angles))
    return pe


_NBUF = 4


def _sc_body(inp_hbm, pe_hbm, table_hbm, out_hbm,
             idx_v, pe_v, rows_v, outs_v, gsems, ssems):
    wid = lax.axis_index("s") * 2 + lax.axis_index("c")
    base = wid * _CPW
    pltpu.sync_copy(inp_hbm.at[pl.ds(base, _CPW)], idx_v)
    pltpu.sync_copy(pe_hbm, pe_v)

    def gather(j, b):
        pltpu.async_copy(table_hbm.at[idx_v.at[j]], rows_v.at[b], gsems.at[b])

    def gather_wait(j, b):
        pltpu.make_async_copy(
            table_hbm.at[idx_v.at[j]], rows_v.at[b], gsems.at[b]).wait()

    def writeout(j, b):
        pltpu.async_copy(outs_v.at[b], out_hbm.at[base + j], ssems.at[b])

    def writeout_wait(j, b):
        pltpu.make_async_copy(
            outs_v.at[b], out_hbm.at[base + j], ssems.at[b]).wait()

    def add(b, p):
        def add_body(r, c):
            for d in range(_DIM // _LANES):
                sl = pl.ds(d * _LANES, _LANES)
                outs_v[b, r, sl] = rows_v[b, r, sl] + pe_v[p * _HALF + r, sl]
            return c
        lax.fori_loop(0, _HALF, add_body, 0, unroll=4)

    for b in range(_NBUF):
        gather(b, b)

    def group(g, carry):
        for b in range(_NBUF):
            j = g * _NBUF + b
            gather_wait(j, b)
            @pl.when(g > 0)
            def _():
                writeout_wait(j - _NBUF, b)
            add(b, b % 2)
            writeout(j, b)
            @pl.when(g < _CPW // _NBUF - 1)
            def _():
                gather(j + _NBUF, b)
        return carry

    lax.fori_loop(0, _CPW // _NBUF, group, 0)

    for b in range(_NBUF):
        writeout_wait(_CPW - _NBUF + b, b)


def kernel(inp, table):
    pe = _pe_table()
    inp2 = inp.reshape(_ROWS, _HALF).astype(jnp.int32)
    mesh = plsc.VectorSubcoreMesh(core_axis_name="c", subcore_axis_name="s")
    run = pl.kernel(
        _sc_body,
        out_type=jax.ShapeDtypeStruct((_ROWS, _HALF, _DIM), jnp.float32),
        mesh=mesh,
        scratch_types=[
            pltpu.VMEM((_CPW, _HALF), jnp.int32),
            pltpu.VMEM((_L, _DIM), jnp.float32),
            pltpu.VMEM((_NBUF, _HALF, _DIM), jnp.float32),
            pltpu.VMEM((_NBUF, _HALF, _DIM), jnp.float32),
            pltpu.SemaphoreType.DMA((_NBUF,)),
            pltpu.SemaphoreType.DMA((_NBUF,)),
        ],
        compiler_params=pltpu.CompilerParams(use_tc_tiling_on_sc=False),
    )
    out = run(inp2, pe, table)
    return out.reshape(_B, _L, _DIM)


# speedup vs baseline: 1.1471x; 1.0407x over previous
import jax
import jax.numpy as jnp
from jax import lax
from jax.experimental import pallas as pl
from jax.experimental.pallas import tpu as pltpu
from jax.experimental.pallas import tpu_sc as plsc

_VOCAB = 1000000
_DIM = 64
_B = 1024
_L = 200

_HALF = 100
_ROWS = _B * _L // _HALF
_NW = 32
_CPW = _ROWS // _NW
_LANES = 16


def _pe_table():
    pos = jnp.arange(_L, dtype=jnp.float32)[:, None]
    i = jnp.arange(0, _DIM, 2, dtype=jnp.float32)
    div = jnp.exp(-jnp.log(10000.0) * i / _DIM)
    angles = pos * div[None, :]
    pe = jnp.zeros((_L, _DIM), dtype=jnp.float32)
    pe = pe.at[:, 0::2].set(jnp.sin(angles))
    pe = pe.at[:, 1::2].set(jnp.cos(angles))
    return pe


_NBUF = 8


def _sc_body(inp_hbm, pe_hbm, table_hbm, out_hbm,
             idx_v, pe_v, rows_v, outs_v, gsems, ssems):
    wid = lax.axis_index("s") * 2 + lax.axis_index("c")
    base = wid * _CPW
    pltpu.sync_copy(inp_hbm.at[pl.ds(base, _CPW)], idx_v)
    pltpu.sync_copy(pe_hbm, pe_v)

    def gather(j, b):
        pltpu.async_copy(table_hbm.at[idx_v.at[j]], rows_v.at[b], gsems.at[b])

    def gather_wait(j, b):
        pltpu.make_async_copy(
            table_hbm.at[idx_v.at[j]], rows_v.at[b], gsems.at[b]).wait()

    def writeout(j, b):
        pltpu.async_copy(outs_v.at[b], out_hbm.at[base + j], ssems.at[b])

    def writeout_wait(j, b):
        pltpu.make_async_copy(
            outs_v.at[b], out_hbm.at[base + j], ssems.at[b]).wait()

    def add(b, p):
        def add_body(r, c):
            for d in range(_DIM // _LANES):
                sl = pl.ds(d * _LANES, _LANES)
                outs_v[b, r, sl] = rows_v[b, r, sl] + pe_v[p * _HALF + r, sl]
            return c
        lax.fori_loop(0, _HALF, add_body, 0, unroll=10)

    for b in range(_NBUF):
        gather(b, b)

    def group(g, carry):
        for b in range(_NBUF):
            j = g * _NBUF + b
            gather_wait(j, b)
            @pl.when(g > 0)
            def _():
                writeout_wait(j - _NBUF, b)
            add(b, b % 2)
            writeout(j, b)
            @pl.when(g < _CPW // _NBUF - 1)
            def _():
                gather(j + _NBUF, b)
        return carry

    lax.fori_loop(0, _CPW // _NBUF, group, 0)

    for b in range(_NBUF):
        writeout_wait(_CPW - _NBUF + b, b)


def kernel(inp, table):
    pe = _pe_table()
    inp2 = inp.reshape(_ROWS, _HALF).astype(jnp.int32)
    mesh = plsc.VectorSubcoreMesh(core_axis_name="c", subcore_axis_name="s")
    run = pl.kernel(
        _sc_body,
        out_type=jax.ShapeDtypeStruct((_ROWS, _HALF, _DIM), jnp.float32),
        mesh=mesh,
        scratch_types=[
            pltpu.VMEM((_CPW, _HALF), jnp.int32),
            pltpu.VMEM((_L, _DIM), jnp.float32),
            pltpu.VMEM((_NBUF, _HALF, _DIM), jnp.float32),
            pltpu.VMEM((_NBUF, _HALF, _DIM), jnp.float32),
            pltpu.SemaphoreType.DMA((_NBUF,)),
            pltpu.SemaphoreType.DMA((_NBUF,)),
        ],
        compiler_params=pltpu.CompilerParams(use_tc_tiling_on_sc=False),
    )
    out = run(inp2, pe, table)
    return out.reshape(_B, _L, _DIM)
